# baseline (speedup 1.0000x reference)
import dataclasses
import functools

import jax
import jax.numpy as jnp
from jax import lax
from jax.experimental import pallas as pl
from jax.experimental.pallas import tpu as pltpu
from jax.experimental.pallas import tpu_sc as plsc

N = 10000
E = 320000
D = 128

NC = 2
NS = 16
NW = NC * NS
CHUNK = 128
SEG = 40
NSEG = 2
ITERS = SEG * NSEG
EPAD = NW * ITERS * CHUNK
NPAD = 10240
ROWS_PER_TILE = NPAD // NS
ZBLK = CHUNK
NBLK = ROWS_PER_TILE // ZBLK


def _sc_aggregate(feature, src_idx, dst_idx):
    mesh = plsc.VectorSubcoreMesh(core_axis_name="c", subcore_axis_name="s")

    @functools.partial(
        pl.kernel,
        out_type=jax.ShapeDtypeStruct((NC, NPAD, D), jnp.float32),
        mesh=mesh,
        scratch_types=[
            pltpu.VMEM_SHARED((NPAD, D), jnp.float32),
            pltpu.VMEM((SEG, CHUNK), jnp.int32),
            pltpu.VMEM((SEG, CHUNK), jnp.int32),
            pltpu.VMEM((CHUNK, D), jnp.float32),
            pltpu.VMEM((CHUNK, D), jnp.float32),
            pltpu.SemaphoreType.DMA,
            pltpu.SemaphoreType.DMA,
        ],
    )
    def k(feat_hbm, src_hbm, dst_hbm, pacc_hbm,
          acc_sh, src_v, dst_v, rows0_v, rows1_v, gsem, ssem):
        c = lax.axis_index("c")
        s = lax.axis_index("s")
        w = c * NS + s
        row0 = s * ROWS_PER_TILE

        def wait_g(buf):
            pltpu.make_async_copy(feat_hbm.at[pl.ds(0, CHUNK)], buf,
                                  gsem).wait()

        def wait_s(buf):
            pltpu.make_async_copy(feat_hbm.at[pl.ds(0, CHUNK)], buf,
                                  ssem).wait()

        def fill(buf, val):
            @pl.loop(0, ZBLK)
            def _(r):
                @pl.loop(0, D, step=16)
                def _(col):
                    buf.at[pl.ds(r, 1), pl.ds(col, 16)][...] = jnp.full(
                        (1, 16), val, jnp.float32)

        fill(rows0_v, 0.0)
        for kblk in range(NBLK):
            pltpu.async_copy(rows0_v,
                             acc_sh.at[pl.ds(row0 + kblk * ZBLK, ZBLK)],
                             ssem)
        for kblk in range(NBLK):
            wait_s(rows0_v)
        plsc.subcore_barrier()

        @pl.loop(0, NSEG)
        def _(g):
            pltpu.sync_copy(src_hbm.at[w].at[g], src_v)
            pltpu.sync_copy(dst_hbm.at[w].at[g], dst_v)
            pltpu.async_copy(feat_hbm.at[src_v.at[0]], rows0_v, gsem)

            @pl.loop(0, SEG, step=2)
            def _(i):
                @pl.when(i > 0)
                def _():
                    wait_s(rows1_v)
                wait_g(rows0_v)
                pltpu.async_copy(rows0_v, acc_sh.at[dst_v.at[i]], ssem,
                                 add=True)
                pltpu.async_copy(feat_hbm.at[src_v.at[i + 1]], rows1_v,
                                 gsem)
                wait_s(rows0_v)
                wait_g(rows1_v)
                pltpu.async_copy(rows1_v, acc_sh.at[dst_v.at[i + 1]], ssem,
                                 add=True)

                @pl.when(i + 2 < SEG)
                def _():
                    pltpu.async_copy(feat_hbm.at[src_v.at[i + 2]], rows0_v,
                                     gsem)

            wait_s(rows1_v)

        plsc.subcore_barrier()
        for kblk in range(NBLK):
            r0 = row0 + kblk * ZBLK
            pltpu.sync_copy(acc_sh.at[pl.ds(r0, ZBLK)], rows0_v)
            pltpu.sync_copy(rows0_v, pacc_hbm.at[c].at[pl.ds(r0, ZBLK)])

    return k(feature, src_idx, dst_idx)


def _sc_count_hist(dst_flat):
    mesh = plsc.VectorSubcoreMesh(core_axis_name="c", subcore_axis_name="s")
    cp = pltpu.CompilerParams()
    if "needs_layout_passes" in pltpu.CompilerParams.__dataclass_fields__:
        cp = dataclasses.replace(cp, needs_layout_passes=False)

    @functools.partial(
        pl.kernel,
        out_type=jax.ShapeDtypeStruct((NW, NPAD), jnp.float32),
        mesh=mesh,
        compiler_params=cp,
        scratch_types=[
            pltpu.VMEM((SEG * CHUNK,), jnp.int32),
            pltpu.VMEM((NPAD,), jnp.float32),
        ],
    )
    def k(dst_hbm, pcnt_hbm, dst_f, hist_v):
        c = lax.axis_index("c")
        s = lax.axis_index("s")
        w = c * NS + s

        @pl.loop(0, NPAD, step=16)
        def _(j):
            hist_v.at[pl.ds(j, 16)][...] = jnp.zeros((16,), jnp.float32)

        ones16 = jnp.ones((16,), jnp.float32)

        @pl.loop(0, NSEG)
        def _(g):
            pltpu.sync_copy(dst_hbm.at[w].at[g], dst_f)

            @pl.loop(0, SEG * CHUNK, step=16)
            def _(j):
                idx16 = dst_f.at[pl.ds(j, 16)][...]
                plsc.addupdate_scatter(hist_v, [idx16], ones16)

        pltpu.sync_copy(hist_v, pcnt_hbm.at[w])

    return k(dst_flat)


def _tc_combine(pacc, pcnt, feature, W_l, b_l, W_r):
    def body(pacc_ref, pcnt_ref, feat_ref, wl_ref, bl_ref, wr_ref,
             out_ref):
        acc = pacc_ref[0] + pacc_ref[1]
        cnt = jnp.sum(pcnt_ref[...], axis=0).reshape(NPAD, 1)
        mean = (acc / jnp.maximum(cnt, 1.0))[:N]
        out_ref[...] = (
            jnp.dot(mean, wl_ref[...], preferred_element_type=jnp.float32)
            + bl_ref[...]
            + jnp.dot(feat_ref[...], wr_ref[...],
                      preferred_element_type=jnp.float32)
        )

    return pl.pallas_call(
        body,
        out_shape=jax.ShapeDtypeStruct((N, D), jnp.float32),
    )(pacc, pcnt, feature, W_l, b_l, W_r)


def kernel(feature, edge_index, W_l, b_l, W_r):
    npad_idx = jnp.arange(EPAD - E, dtype=jnp.int32)
    src = jnp.concatenate([edge_index[0], npad_idx % N])
    dst = jnp.concatenate([edge_index[1], N + npad_idx % (NPAD - N)])
    src = src.reshape(NW, NSEG, SEG, CHUNK)
    dst = dst.reshape(NW, NSEG, SEG, CHUNK)
    pacc = _sc_aggregate(feature, src, dst)
    pcnt = _sc_count_hist(dst.reshape(NW, NSEG, SEG * CHUNK))
    return _tc_combine(pacc, pcnt, feature, W_l, b_l.reshape(1, D), W_r)
